# baseline (speedup 1.0000x reference)
import functools

import jax
import jax.numpy as jnp
from jax import lax
from jax.experimental import pallas as pl
from jax.experimental.pallas import tpu as pltpu
from jax.experimental.pallas import tpu_sc as plsc

N_ROWS = 1_000_000
N_COLS = 64
VR = 500_000
VC = 128
NUM_CORES = 2
NUM_SUBCORES = 16
NW = NUM_CORES * NUM_SUBCORES
ROWS_W = 15_624
R = 120
N_MAIN = 130
TAIL1_R = ROWS_W - N_MAIN * R
TAIL2_0 = NW * ROWS_W
TAIL2_R = 8
N_GROUPS = N_MAIN // 2

_mesh = plsc.VectorSubcoreMesh(core_axis_name="c", subcore_axis_name="s")


def _mul_rows(ib, ob, n_rows):
    def body(r, carry):
        for k in range(VC // 16):
            sl = pl.ds(k * 16, 16)
            ob[r, sl] = ib[r, sl] * 10.0
        return carry

    lax.fori_loop(0, n_rows, body, 0)


@functools.partial(
    pl.kernel,
    mesh=_mesh,
    out_type=jax.ShapeDtypeStruct((VR, VC), jnp.float32),
    scratch_types=[
        pltpu.VMEM((R, VC), jnp.float32),
        pltpu.VMEM((R, VC), jnp.float32),
        pltpu.VMEM((R, VC), jnp.float32),
        pltpu.VMEM((R, VC), jnp.float32),
        pltpu.SemaphoreType.DMA,
        pltpu.SemaphoreType.DMA,
        pltpu.SemaphoreType.DMA,
        pltpu.SemaphoreType.DMA,
    ],
)
def _mul10(x_hbm, o_hbm, ib0, ib1, ob0, ob1, is0, is1, os0, os1):
    wid = lax.axis_index("s") * NUM_CORES + lax.axis_index("c")
    base = pl.multiple_of(wid * ROWS_W, 8)
    ibufs, obufs = (ib0, ib1), (ob0, ob1)
    isems, osems = (is0, is1), (os0, os1)

    def row0_of(c):
        return pl.multiple_of(base + c * R, 8)

    pltpu.async_copy(x_hbm.at[pl.ds(row0_of(0), R), :], ib0, is0)
    pltpu.async_copy(x_hbm.at[pl.ds(row0_of(1), R), :], ib1, is1)

    def group_body(g, carry):
        for b in range(2):
            c = g * 2 + b
            row0 = row0_of(c)
            ib, ob = ibufs[b], obufs[b]
            pltpu.make_async_copy(x_hbm.at[pl.ds(row0, R), :], ib, isems[b]).wait()

            @pl.when(g >= 1)
            def _wait_out():
                pltpu.make_async_copy(ob, o_hbm.at[pl.ds(row0, R), :], osems[b]).wait()

            _mul_rows(ib, ob, R)

            pltpu.async_copy(ob, o_hbm.at[pl.ds(row0, R), :], osems[b])

            @pl.when(g < N_GROUPS - 1)
            def _next_in():
                pltpu.async_copy(
                    x_hbm.at[pl.ds(row0_of(c + 2), R), :], ib, isems[b]
                )

        return carry

    lax.fori_loop(0, N_GROUPS, group_body, 0)

    tail1_row = row0_of(N_MAIN)
    tail2_row = pl.multiple_of(TAIL2_0 + wid * TAIL2_R, 8)
    is_tail2_w = wid < 4

    pltpu.async_copy(
        x_hbm.at[pl.ds(tail1_row, TAIL1_R), :], ib0.at[pl.ds(0, TAIL1_R), :], is0
    )

    @pl.when(is_tail2_w)
    def _tail2_in():
        pltpu.async_copy(
            x_hbm.at[pl.ds(tail2_row, TAIL2_R), :], ib1.at[pl.ds(0, TAIL2_R), :], is1
        )

    rl0, rl1 = row0_of(N_MAIN - 2), row0_of(N_MAIN - 1)
    pltpu.make_async_copy(ob0, o_hbm.at[pl.ds(rl0, R), :], os0).wait()
    pltpu.make_async_copy(ob1, o_hbm.at[pl.ds(rl1, R), :], os1).wait()

    pltpu.make_async_copy(
        x_hbm.at[pl.ds(tail1_row, TAIL1_R), :], ib0.at[pl.ds(0, TAIL1_R), :], is0
    ).wait()
    _mul_rows(ib0, ob0, TAIL1_R)
    pltpu.async_copy(
        ob0.at[pl.ds(0, TAIL1_R), :], o_hbm.at[pl.ds(tail1_row, TAIL1_R), :], os0
    )

    @pl.when(is_tail2_w)
    def _tail2_work():
        pltpu.make_async_copy(
            x_hbm.at[pl.ds(tail2_row, TAIL2_R), :], ib1.at[pl.ds(0, TAIL2_R), :], is1
        ).wait()
        _mul_rows(ib1, ob1, TAIL2_R)
        pltpu.async_copy(
            ob1.at[pl.ds(0, TAIL2_R), :], o_hbm.at[pl.ds(tail2_row, TAIL2_R), :], os1
        )
        pltpu.make_async_copy(
            ob1.at[pl.ds(0, TAIL2_R), :], o_hbm.at[pl.ds(tail2_row, TAIL2_R), :], os1
        ).wait()

    pltpu.make_async_copy(
        ob0.at[pl.ds(0, TAIL1_R), :], o_hbm.at[pl.ds(tail1_row, TAIL1_R), :], os0
    ).wait()


def kernel(input_tensor):
    view = input_tensor.reshape(VR, VC)
    return _mul10(view).reshape(N_ROWS, N_COLS)


# speedup vs baseline: 5.0085x; 5.0085x over previous
import functools

import jax
import jax.numpy as jnp
from jax import lax
from jax.experimental import pallas as pl
from jax.experimental.pallas import tpu as pltpu
from jax.experimental.pallas import tpu_sc as plsc

N_ROWS = 1_000_000
N_COLS = 64
NUM_CORES = 2
NUM_SUBCORES = 16
NW = NUM_CORES * NUM_SUBCORES
RB = 8
W = 1024
N_MAIN = 244
N_GROUPS = N_MAIN // 2
MINI0 = N_MAIN * 4 * W
MINI_W = 128
TAIL0 = MINI0 + 4 * MINI_W
TAIL_W = 64

_mesh = plsc.VectorSubcoreMesh(core_axis_name="c", subcore_axis_name="s")


def _mul_cols(ib, ob, n_vecs):
    def body(k, carry):
        sl = pl.ds(k * 16, 16)
        for r in range(RB):
            ob[r, sl] = ib[r, sl] * 10.0
        return carry

    lax.fori_loop(0, n_vecs, body, 0)


@functools.partial(
    pl.kernel,
    mesh=_mesh,
    out_type=jax.ShapeDtypeStruct((N_COLS, N_ROWS), jnp.float32),
    scratch_types=[
        pltpu.VMEM((RB, W), jnp.float32),
        pltpu.VMEM((RB, W), jnp.float32),
        pltpu.VMEM((RB, W), jnp.float32),
        pltpu.VMEM((RB, W), jnp.float32),
        pltpu.SemaphoreType.DMA,
        pltpu.SemaphoreType.DMA,
        pltpu.SemaphoreType.DMA,
        pltpu.SemaphoreType.DMA,
    ],
)
def _mul10(x_hbm, o_hbm, ib0, ib1, ob0, ob1, is0, is1, os0, os1):
    wid = lax.axis_index("s") * NUM_CORES + lax.axis_index("c")
    tr = wid // 4
    q = wid % 4
    row0 = pl.multiple_of(tr * RB, 8)
    rows = pl.ds(row0, RB)
    ibufs, obufs = (ib0, ib1), (ob0, ob1)
    isems, osems = (is0, is1), (os0, os1)

    def col0_of(c):
        return pl.multiple_of((q * N_MAIN + c) * W, 128)

    pltpu.async_copy(x_hbm.at[rows, pl.ds(col0_of(0), W)], ib0, is0)
    pltpu.async_copy(x_hbm.at[rows, pl.ds(col0_of(1), W)], ib1, is1)

    def group_body(g, carry):
        for b in range(2):
            c = g * 2 + b
            col0 = col0_of(c)
            ib, ob = ibufs[b], obufs[b]
            pltpu.make_async_copy(x_hbm.at[rows, pl.ds(col0, W)], ib, isems[b]).wait()

            @pl.when(g >= 1)
            def _wait_out():
                pltpu.make_async_copy(
                    ob, o_hbm.at[rows, pl.ds(col0, W)], osems[b]
                ).wait()

            _mul_cols(ib, ob, W // 16)

            pltpu.async_copy(ob, o_hbm.at[rows, pl.ds(col0, W)], osems[b])

            @pl.when(g < N_GROUPS - 1)
            def _next_in():
                pltpu.async_copy(
                    x_hbm.at[rows, pl.ds(col0_of(c + 2), W)], ib, isems[b]
                )

        return carry

    lax.fori_loop(0, N_GROUPS, group_body, 0)

    mini_col = pl.multiple_of(MINI0 + q * MINI_W, 128)

    pltpu.async_copy(
        x_hbm.at[rows, pl.ds(mini_col, MINI_W)], ib0.at[:, pl.ds(0, MINI_W)], is0
    )

    cl0, cl1 = col0_of(N_MAIN - 2), col0_of(N_MAIN - 1)
    pltpu.make_async_copy(ob0, o_hbm.at[rows, pl.ds(cl0, W)], os0).wait()
    pltpu.make_async_copy(ob1, o_hbm.at[rows, pl.ds(cl1, W)], os1).wait()

    pltpu.make_async_copy(
        x_hbm.at[rows, pl.ds(mini_col, MINI_W)], ib0.at[:, pl.ds(0, MINI_W)], is0
    ).wait()
    _mul_cols(ib0, ob0, MINI_W // 16)
    pltpu.async_copy(
        ob0.at[:, pl.ds(0, MINI_W)], o_hbm.at[rows, pl.ds(mini_col, MINI_W)], os0
    )
    pltpu.make_async_copy(
        ob0.at[:, pl.ds(0, MINI_W)], o_hbm.at[rows, pl.ds(mini_col, MINI_W)], os0
    ).wait()


def kernel(input_tensor):
    big = _mul10(input_tensor.T).T
    tail = input_tensor[TAIL0:, :] * 10.0
    return lax.dynamic_update_slice(big, tail, (TAIL0, 0))


# speedup vs baseline: 6.9784x; 1.3933x over previous
import functools

import jax
import jax.numpy as jnp
from jax import lax
from jax.experimental import pallas as pl
from jax.experimental.pallas import tpu as pltpu
from jax.experimental.pallas import tpu_sc as plsc

N_ROWS = 1_000_000
N_COLS = 64
NUM_CORES = 2
NUM_SUBCORES = 16
NW = NUM_CORES * NUM_SUBCORES
RB = 8
W = 1920
N_MAIN = 130
N_GROUPS = N_MAIN // 2
MINI0 = N_MAIN * 4 * W
MINI_W = 384
TAIL0 = MINI0 + 4 * MINI_W
TAIL_W = 64

_mesh = plsc.VectorSubcoreMesh(core_axis_name="c", subcore_axis_name="s")


def _mul_cols(ib, ob, n_vecs):
    @plsc.parallel_loop(0, n_vecs, unroll=4)
    def body(k):
        sl = pl.ds(k * 16, 16)
        for r in range(RB):
            ob[r, sl] = ib[r, sl] * 10.0


@functools.partial(
    pl.kernel,
    mesh=_mesh,
    out_type=jax.ShapeDtypeStruct((N_COLS, N_ROWS), jnp.float32),
    scratch_types=[
        pltpu.VMEM((RB, W), jnp.float32),
        pltpu.VMEM((RB, W), jnp.float32),
        pltpu.VMEM((RB, W), jnp.float32),
        pltpu.VMEM((RB, W), jnp.float32),
        pltpu.SemaphoreType.DMA,
        pltpu.SemaphoreType.DMA,
        pltpu.SemaphoreType.DMA,
        pltpu.SemaphoreType.DMA,
    ],
)
def _mul10(x_hbm, o_hbm, ib0, ib1, ob0, ob1, is0, is1, os0, os1):
    wid = lax.axis_index("s") * NUM_CORES + lax.axis_index("c")
    tr = wid // 4
    q = wid % 4
    row0 = pl.multiple_of(tr * RB, 8)
    rows = pl.ds(row0, RB)
    ibufs, obufs = (ib0, ib1), (ob0, ob1)
    isems, osems = (is0, is1), (os0, os1)

    def col0_of(c):
        return pl.multiple_of((q * N_MAIN + c) * W, 128)

    pltpu.async_copy(x_hbm.at[rows, pl.ds(col0_of(0), W)], ib0, is0)
    pltpu.async_copy(x_hbm.at[rows, pl.ds(col0_of(1), W)], ib1, is1)

    def group_body(g, carry):
        for b in range(2):
            c = g * 2 + b
            col0 = col0_of(c)
            ib, ob = ibufs[b], obufs[b]
            pltpu.make_async_copy(x_hbm.at[rows, pl.ds(col0, W)], ib, isems[b]).wait()

            @pl.when(g >= 1)
            def _wait_out():
                pltpu.make_async_copy(
                    ob, o_hbm.at[rows, pl.ds(col0, W)], osems[b]
                ).wait()

            _mul_cols(ib, ob, W // 16)

            pltpu.async_copy(ob, o_hbm.at[rows, pl.ds(col0, W)], osems[b])

            @pl.when(g < N_GROUPS - 1)
            def _next_in():
                pltpu.async_copy(
                    x_hbm.at[rows, pl.ds(col0_of(c + 2), W)], ib, isems[b]
                )

        return carry

    lax.fori_loop(0, N_GROUPS, group_body, 0)

    mini_col = pl.multiple_of(MINI0 + q * MINI_W, 128)

    pltpu.async_copy(
        x_hbm.at[rows, pl.ds(mini_col, MINI_W)], ib0.at[:, pl.ds(0, MINI_W)], is0
    )

    cl0, cl1 = col0_of(N_MAIN - 2), col0_of(N_MAIN - 1)
    pltpu.make_async_copy(ob0, o_hbm.at[rows, pl.ds(cl0, W)], os0).wait()
    pltpu.make_async_copy(ob1, o_hbm.at[rows, pl.ds(cl1, W)], os1).wait()

    pltpu.make_async_copy(
        x_hbm.at[rows, pl.ds(mini_col, MINI_W)], ib0.at[:, pl.ds(0, MINI_W)], is0
    ).wait()
    _mul_cols(ib0, ob0, MINI_W // 16)
    pltpu.async_copy(
        ob0.at[:, pl.ds(0, MINI_W)], o_hbm.at[rows, pl.ds(mini_col, MINI_W)], os0
    )
    pltpu.make_async_copy(
        ob0.at[:, pl.ds(0, MINI_W)], o_hbm.at[rows, pl.ds(mini_col, MINI_W)], os0
    ).wait()


def kernel(input_tensor):
    big = _mul10(input_tensor.T).T
    tail = input_tensor[TAIL0:, :] * 10.0
    return lax.dynamic_update_slice(big, tail, (TAIL0, 0))
